# Initial kernel scaffold; baseline (speedup 1.0000x reference)
import jax
import jax.numpy as jnp
from jax.experimental import pallas as pl


def kernel(hidden_states, gate_w, w1, w3, w2):
    raise NotImplementedError("write your pallas kernel here")


# speedup vs baseline: 1.4861x; 1.4861x over previous
import functools

import jax
import jax.numpy as jnp
from jax.experimental import pallas as pl
from jax.experimental.pallas import tpu as pltpu

S, D, FF, E = 2048, 1024, 3584, 8
BF = 512
NJ = FF // BF


def _router_body(x_ref, g_ref, logits_ref, wsum_ref):
    x = x_ref[...]
    g = g_ref[...]
    logits = jnp.dot(x, g, preferred_element_type=jnp.float32)
    logits_ref[...] = logits
    lane = jax.lax.broadcasted_iota(jnp.int32, logits.shape, 1)
    m1 = jnp.max(logits, axis=1, keepdims=True)
    i1 = jnp.min(jnp.where(logits == m1, lane, E), axis=1, keepdims=True)
    mask1 = lane == i1
    l2 = jnp.max(jnp.where(mask1, -jnp.inf, logits), axis=1, keepdims=True)
    i2 = jnp.min(
        jnp.where((logits == l2) & (~mask1), lane, E), axis=1, keepdims=True
    )
    p1 = jax.nn.sigmoid(m1 - l2)
    p2 = jax.nn.sigmoid(l2 - m1)
    wsum = jnp.where(mask1, p1, 0.0) + jnp.where(lane == i2, p2, 0.0)
    wsum_ref[...] = wsum


def _moe_body(wsum_ref, x_ref, w1_ref, w3_ref, w2_ref, out_ref):
    e = pl.program_id(0)
    j = pl.program_id(1)

    @pl.when((e == 0) & (j == 0))
    def _init():
        out_ref[...] = jnp.zeros_like(out_ref)

    x = x_ref[...]
    h1 = jnp.dot(x, w1_ref[0], preferred_element_type=jnp.float32)
    h1 = h1 * jax.nn.sigmoid(h1)
    h3 = jnp.dot(x, w3_ref[0], preferred_element_type=jnp.float32)
    y = jnp.dot(h1 * h3, w2_ref[0], preferred_element_type=jnp.float32)
    lane = jax.lax.broadcasted_iota(jnp.int32, (S, E), 1)
    w_e = jnp.sum(
        jnp.where(lane == e, wsum_ref[...], 0.0), axis=1, keepdims=True
    )
    out_ref[...] += y * w_e


@jax.jit
def kernel(hidden_states, gate_w, w1, w3, w2):
    x = hidden_states.reshape(S, D)

    logits, wsum = pl.pallas_call(
        _router_body,
        grid=(1,),
        in_specs=[
            pl.BlockSpec((S, D), lambda i: (0, 0)),
            pl.BlockSpec((D, E), lambda i: (0, 0)),
        ],
        out_specs=[
            pl.BlockSpec((S, E), lambda i: (0, 0)),
            pl.BlockSpec((S, E), lambda i: (0, 0)),
        ],
        out_shape=[
            jax.ShapeDtypeStruct((S, E), jnp.float32),
            jax.ShapeDtypeStruct((S, E), jnp.float32),
        ],
    )(x, gate_w)

    out = pl.pallas_call(
        _moe_body,
        grid=(E, NJ),
        in_specs=[
            pl.BlockSpec((S, E), lambda e, j: (0, 0)),
            pl.BlockSpec((S, D), lambda e, j: (0, 0)),
            pl.BlockSpec((1, D, BF), lambda e, j: (e, 0, j)),
            pl.BlockSpec((1, D, BF), lambda e, j: (e, 0, j)),
            pl.BlockSpec((1, BF, D), lambda e, j: (e, j, 0)),
        ],
        out_specs=pl.BlockSpec((S, D), lambda e, j: (0, 0)),
        out_shape=jax.ShapeDtypeStruct((S, D), jnp.float32),
        compiler_params=pltpu.CompilerParams(
            dimension_semantics=("arbitrary", "arbitrary"),
        ),
    )(wsum, x, w1, w3, w2)

    return out.reshape(1, S, D), logits.reshape(1, S, E)
